# baseline (speedup 1.0000x reference)
import math

import jax
import jax.numpy as jnp
from jax.experimental import pallas as pl
from jax.experimental.pallas import tpu as pltpu

_NUM_SEGMENTS = 16
_NCHUNKS = 4


def _seg_softmax_body(x_hbm, seg_ref, w_ref, out_ref, x_vmem, sems):
    n, d = x_vmem.shape
    chunk = n // _NCHUNKS
    for i in range(_NCHUNKS):
        pltpu.make_async_copy(
            x_hbm.at[pl.ds(i * chunk, chunk), :],
            x_vmem.at[pl.ds(i * chunk, chunk), :],
            sems.at[i]).start()
    seg = seg_ref[...]
    w = w_ref[...]
    onehot = (seg == jax.lax.broadcasted_iota(
        jnp.int32, (_NUM_SEGMENTS, 1), 0)).astype(jnp.float32)
    es = []
    for i in range(_NCHUNKS):
        pltpu.make_async_copy(
            x_hbm.at[pl.ds(i * chunk, chunk), :],
            x_vmem.at[pl.ds(i * chunk, chunk), :],
            sems.at[i]).wait()
        xi = x_vmem[pl.ds(i * chunk, chunk), :]
        si = jax.lax.dot_general(w, xi, (((0,), (1,)), ((), ())),
                                 preferred_element_type=jnp.float32)
        es.append(si)
    s = jnp.concatenate(es, axis=1)
    gmax = jnp.max(s, axis=1, keepdims=True)
    e = jnp.exp(s - gmax)
    denom = jax.lax.dot_general(e, onehot, (((1,), (1,)), ((), ())),
                                preferred_element_type=jnp.float32)
    d_tok = jnp.dot(denom, onehot,
                    preferred_element_type=jnp.float32)
    out_ref[...] = e / d_tok


def kernel(inputs, segment_ids, lengths, W1, b1, W2, b2, W3, b3, Wr, br,
           W_k, W_q):
    del lengths, W1, b1, W2, b2, W3, b3, Wr, br
    n, d = inputs.shape
    h, dp = W_q.shape
    w_eff = jnp.einsum('dhp,hp->dh', W_k[:d].reshape(d, h, dp),
                       W_q) / math.sqrt(dp)
    seg = segment_ids.astype(jnp.int32).reshape(1, n)
    out = pl.pallas_call(
        _seg_softmax_body,
        in_specs=[pl.BlockSpec(memory_space=pltpu.MemorySpace.HBM),
                  pl.BlockSpec(memory_space=pltpu.MemorySpace.VMEM),
                  pl.BlockSpec(memory_space=pltpu.MemorySpace.VMEM)],
        out_shape=jax.ShapeDtypeStruct((h, n), jnp.float32),
        scratch_shapes=[pltpu.VMEM((n, d), jnp.float32),
                        pltpu.SemaphoreType.DMA((_NCHUNKS,))],
    )(inputs, seg, w_eff)
    return out[:, :, None]
